# baseline (speedup 1.0000x reference)
import functools

import jax
import jax.numpy as jnp
from jax import lax
from jax.experimental import pallas as pl
from jax.experimental.pallas import tpu as pltpu
from jax.experimental.pallas import tpu_sc as plsc

N_TOK = 8192
N_EMB = 8192
D = 256
BLK = 128
GRID = N_TOK // BLK

NC, NS = 2, 16
NW = NC * NS
BPW = N_TOK // NW
CH = 128
NCH = BPW // CH


def _vq_body(x_ref, se_ref, emb_ref,
             enc_ref, idx_ref, loss_ref, perp_ref,
             counts_ref, sse_ref):
    i = pl.program_id(0)
    xb = x_ref[...]
    emb = emb_ref[...]

    mm = jax.lax.dot_general(xb, emb, (((1,), (1,)), ((), ())),
                             preferred_element_type=jnp.float32)
    sxb = jnp.sum(xb * xb, axis=1, keepdims=True)

    CW = 128
    colf = jax.lax.broadcasted_iota(jnp.int32, (BLK, CW), 1).astype(jnp.float32)
    mv = mi = None
    for kk in range(N_EMB // CW):
        sl = slice(kk * CW, (kk + 1) * CW)
        d = (sxb + se_ref[:, sl]) - 2.0 * mm[:, sl]
        cf = colf + jnp.float32(kk * CW)
        if mv is None:
            mv, mi = d, cf
        else:
            lt = d < mv
            mv = jnp.minimum(mv, d)
            mi = jnp.where(lt, cf, mi)

    minval = jnp.min(mv, axis=1, keepdims=True)
    cand = jnp.where(mv == minval, mi, jnp.float32(3e38))
    idx = jnp.min(cand, axis=1, keepdims=True).astype(jnp.int32)
    idx_ref[...] = idx

    cols = jax.lax.broadcasted_iota(jnp.int32, (BLK, N_EMB), 1)
    enc = (cols == idx).astype(jnp.float32)
    enc_ref[...] = enc

    @pl.when(i == 0)
    def _init():
        counts_ref[...] = jnp.zeros_like(counts_ref)
        sse_ref[0, 0] = jnp.float32(0.0)

    counts_ref[...] += jnp.sum(enc, axis=0, keepdims=True)
    sse_ref[0, 0] += jnp.sum(minval)

    @pl.when(i == pl.num_programs(0) - 1)
    def _fin():
        m = sse_ref[0, 0] / jnp.float32(N_TOK * D)
        loss_ref[...] = jnp.full((1, 1), m + 0.25 * m, jnp.float32)
        p = counts_ref[...] * jnp.float32(1.0 / N_TOK)
        ent = -jnp.sum(p * jnp.log(p + 1e-10), axis=1, keepdims=True)
        perp_ref[...] = jnp.exp(ent)


def _vq_call(flat, se, embedding):
    return pl.pallas_call(
        _vq_body,
        grid=(GRID,),
        in_specs=[
            pl.BlockSpec((BLK, D), lambda i: (i, 0)),
            pl.BlockSpec((1, N_EMB), lambda i: (0, 0)),
            pl.BlockSpec((N_EMB, D), lambda i: (0, 0)),
        ],
        out_specs=[
            pl.BlockSpec((BLK, N_EMB), lambda i: (i, 0)),
            pl.BlockSpec((BLK, 1), lambda i: (i, 0)),
            pl.BlockSpec((1, 1), lambda i: (0, 0)),
            pl.BlockSpec((1, 1), lambda i: (0, 0)),
        ],
        out_shape=[
            jax.ShapeDtypeStruct((N_TOK, N_EMB), jnp.float32),
            jax.ShapeDtypeStruct((N_TOK, 1), jnp.int32),
            jax.ShapeDtypeStruct((1, 1), jnp.float32),
            jax.ShapeDtypeStruct((1, 1), jnp.float32),
        ],
        scratch_shapes=[
            pltpu.VMEM((1, N_EMB), jnp.float32),
            pltpu.SMEM((1, 1), jnp.float32),
        ],
    )(flat, se, embedding)


def _make_gather():
    @functools.partial(
        pl.kernel,
        mesh=plsc.VectorSubcoreMesh(core_axis_name="c", subcore_axis_name="s"),
        out_type=jax.ShapeDtypeStruct((N_TOK, D), jnp.float32),
        scratch_types=[
            pltpu.VMEM((NCH, CH), jnp.int32),
            pltpu.VMEM((NCH, CH, D), jnp.float32),
            pltpu.SemaphoreType.DMA,
            pltpu.SemaphoreType.DMA,
        ],
    )
    def _gather_k(idx_hbm, table_hbm, out_hbm, idx_v, rows_v, gsem, wsem):
        wid = lax.axis_index("s") * NC + lax.axis_index("c")
        base = wid * BPW
        for j in range(NCH):
            pltpu.sync_copy(idx_hbm.at[pl.ds(base + j * CH, CH)], idx_v.at[j])
        gathers = [pltpu.async_copy(table_hbm.at[idx_v.at[j]], rows_v.at[j], gsem)
                   for j in range(NCH)]
        writes = []
        for j in range(NCH):
            gathers[j].wait()
            writes.append(pltpu.async_copy(
                rows_v.at[j], out_hbm.at[pl.ds(base + j * CH, CH)], wsem))
        for wcp in writes:
            wcp.wait()

    return _gather_k


def kernel(inputs, embedding):
    b, c, h, w = inputs.shape
    x = jnp.transpose(inputs, (0, 2, 3, 1))
    flat = x.reshape(-1, c)
    se = jnp.sum(embedding ** 2, axis=1)[None, :]
    enc, idx, loss, perp = _vq_call(flat, se, embedding)
    qst = _make_gather()(idx.reshape(-1), embedding)
    quantized_out = jnp.transpose(qst.reshape(b, h, w, c), (0, 3, 1, 2))
    return (loss[0, 0], quantized_out, perp[0, 0], enc)


# speedup vs baseline: 1.3932x; 1.3932x over previous
import functools

import jax
import jax.numpy as jnp
from jax import lax
from jax.experimental import pallas as pl
from jax.experimental.pallas import tpu as pltpu
from jax.experimental.pallas import tpu_sc as plsc

N_TOK = 8192
N_EMB = 8192
D = 256
BLK = 256
GRID = N_TOK // BLK

NC, NS = 2, 16
NW = NC * NS
BPW = N_TOK // NW
CH = 128
NCH = BPW // CH


def _vq_body(x_ref, se_ref, emb_ref,
             enc_ref, idx_ref, loss_ref, perp_ref,
             counts_ref, sse_ref):
    i = pl.program_id(0)
    xb = x_ref[...]
    emb = emb_ref[...]

    mm = jax.lax.dot_general(xb, emb, (((1,), (1,)), ((), ())),
                             preferred_element_type=jnp.float32)
    sxb = jnp.sum(xb * xb, axis=1, keepdims=True)

    CW = 128
    colf = jax.lax.broadcasted_iota(jnp.int32, (BLK, CW), 1).astype(jnp.float32)
    mv = mi = None
    for kk in range(N_EMB // CW):
        sl = slice(kk * CW, (kk + 1) * CW)
        d = (sxb + se_ref[:, sl]) - 2.0 * mm[:, sl]
        cf = colf + jnp.float32(kk * CW)
        if mv is None:
            mv, mi = d, cf
        else:
            lt = d < mv
            mv = jnp.minimum(mv, d)
            mi = jnp.where(lt, cf, mi)

    minval = jnp.min(mv, axis=1, keepdims=True)
    cand = jnp.where(mv == minval, mi, jnp.float32(3e38))
    idx = jnp.min(cand, axis=1, keepdims=True).astype(jnp.int32)
    idx_ref[...] = idx

    cols = jax.lax.broadcasted_iota(jnp.int32, (BLK, N_EMB), 1)
    enc = (cols == idx).astype(jnp.float32)
    enc_ref[...] = enc

    @pl.when(i == 0)
    def _init():
        counts_ref[...] = jnp.zeros_like(counts_ref)
        sse_ref[0, 0] = jnp.float32(0.0)

    counts_ref[...] += jnp.sum(enc, axis=0, keepdims=True)
    sse_ref[0, 0] += jnp.sum(minval)

    @pl.when(i == pl.num_programs(0) - 1)
    def _fin():
        m = sse_ref[0, 0] / jnp.float32(N_TOK * D)
        loss_ref[...] = jnp.full((1, 1), m + 0.25 * m, jnp.float32)
        p = counts_ref[...] * jnp.float32(1.0 / N_TOK)
        ent = -jnp.sum(p * jnp.log(p + 1e-10), axis=1, keepdims=True)
        perp_ref[...] = jnp.exp(ent)


def _vq_call(flat, se, embedding):
    return pl.pallas_call(
        _vq_body,
        grid=(GRID,),
        in_specs=[
            pl.BlockSpec((BLK, D), lambda i: (i, 0)),
            pl.BlockSpec((1, N_EMB), lambda i: (0, 0)),
            pl.BlockSpec((N_EMB, D), lambda i: (0, 0)),
        ],
        out_specs=[
            pl.BlockSpec((BLK, N_EMB), lambda i: (i, 0)),
            pl.BlockSpec((BLK, 1), lambda i: (i, 0)),
            pl.BlockSpec((1, 1), lambda i: (0, 0)),
            pl.BlockSpec((1, 1), lambda i: (0, 0)),
        ],
        out_shape=[
            jax.ShapeDtypeStruct((N_TOK, N_EMB), jnp.float32),
            jax.ShapeDtypeStruct((N_TOK, 1), jnp.int32),
            jax.ShapeDtypeStruct((1, 1), jnp.float32),
            jax.ShapeDtypeStruct((1, 1), jnp.float32),
        ],
        scratch_shapes=[
            pltpu.VMEM((1, N_EMB), jnp.float32),
            pltpu.SMEM((1, 1), jnp.float32),
        ],
        compiler_params=pltpu.CompilerParams(
            vmem_limit_bytes=128 * 1024 * 1024,
        ),
    )(flat, se, embedding)


def _make_gather():
    @functools.partial(
        pl.kernel,
        mesh=plsc.VectorSubcoreMesh(core_axis_name="c", subcore_axis_name="s"),
        out_type=jax.ShapeDtypeStruct((N_TOK, D), jnp.float32),
        scratch_types=[
            pltpu.VMEM((NCH, CH), jnp.int32),
            pltpu.VMEM((NCH, CH, D), jnp.float32),
            pltpu.SemaphoreType.DMA,
            pltpu.SemaphoreType.DMA,
        ],
    )
    def _gather_k(idx_hbm, table_hbm, out_hbm, idx_v, rows_v, gsem, wsem):
        wid = lax.axis_index("s") * NC + lax.axis_index("c")
        base = wid * BPW
        for j in range(NCH):
            pltpu.sync_copy(idx_hbm.at[pl.ds(base + j * CH, CH)], idx_v.at[j])
        gathers = [pltpu.async_copy(table_hbm.at[idx_v.at[j]], rows_v.at[j], gsem)
                   for j in range(NCH)]
        writes = []
        for j in range(NCH):
            gathers[j].wait()
            writes.append(pltpu.async_copy(
                rows_v.at[j], out_hbm.at[pl.ds(base + j * CH, CH)], wsem))
        for wcp in writes:
            wcp.wait()

    return _gather_k


def kernel(inputs, embedding):
    b, c, h, w = inputs.shape
    x = jnp.transpose(inputs, (0, 2, 3, 1))
    flat = x.reshape(-1, c)
    se = jnp.sum(embedding ** 2, axis=1)[None, :]
    enc, idx, loss, perp = _vq_call(flat, se, embedding)
    qst = _make_gather()(idx.reshape(-1), embedding)
    quantized_out = jnp.transpose(qst.reshape(b, h, w, c), (0, 3, 1, 2))
    return (loss[0, 0], quantized_out, perp[0, 0], enc)
